# baseline (speedup 1.0000x reference)
import functools

import numpy as np
import jax
import jax.numpy as jnp
from jax import lax
from jax.experimental import pallas as pl
from jax.experimental.pallas import tpu as pltpu
from jax.experimental.pallas import tpu_sc as plsc

B, T, N, SD, E, S = 32, 32, 8, 128, 32, 32
BS = B * T
SN = S * N
NW = 32
RPW = BS * S // NW
BPW = BS // NW
STEPS = RPW // 2


def _threefry2x32(k0, k1, x0, x1):
    rot = ((13, 15, 26, 6), (17, 29, 16, 24))
    ks = [np.uint32(k0), np.uint32(k1),
          np.uint32(k0) ^ np.uint32(k1) ^ np.uint32(0x1BD11BDA)]
    x0 = (x0 + ks[0]).astype(np.uint32)
    x1 = (x1 + ks[1]).astype(np.uint32)
    for i in range(5):
        for r in rot[i % 2]:
            x0 = (x0 + x1).astype(np.uint32)
            x1 = ((x1 << np.uint32(r)) | (x1 >> np.uint32(32 - r))).astype(np.uint32)
            x1 = x1 ^ x0
        x0 = (x0 + ks[(i + 1) % 3]).astype(np.uint32)
        x1 = (x1 + ks[(i + 2) % 3] + np.uint32(i + 1)).astype(np.uint32)
    return x0, x1


def _uniform_key42(shape):
    size = int(np.prod(shape))
    counts = np.arange(size, dtype=np.uint32)
    b0, b1 = _threefry2x32(0, 42, np.zeros(size, np.uint32), counts)
    bits = (b0 ^ b1).reshape(shape)
    f = ((bits >> np.uint32(9)) | np.uint32(0x3F800000)).view(np.float32)
    return np.maximum(0.0, f - 1.0).astype(np.float32)


def _sampling_constants():
    u = _uniform_key42((BS * S, N))
    gc = np.argsort(u, axis=1, kind="stable").astype(np.int32)
    r_idx = np.arange(BS * S)
    b_local = ((r_idx // S) % BPW).astype(np.int32)
    gc_adj = gc + (b_local * N)[:, None]
    seg = (r_idx % 2).astype(np.int32)
    idx = np.maximum(gc - 1, 0) + (seg * 8)[:, None]
    rden = np.where(gc == 0, 0.0, 1.0 / np.maximum(gc, 1)).astype(np.float32)
    sel = np.zeros((SN, N), np.float32)
    for i in range(N):
        sel[i::N, i] = 1.0 / S
    return (gc_adj.reshape(-1), idx.reshape(-1).astype(np.int32),
            rden.reshape(-1), sel)


_GC_ADJ, _IC_IDX, _RDEN, _SEL = _sampling_constants()


def _dyn_gather(x, idx):
    return lax.gather(
        x, idx[:, None],
        dimension_numbers=lax.GatherDimensionNumbers(
            offset_dims=(), collapsed_slice_dims=(0,), start_index_map=(0,)),
        slice_sizes=(1,), mode=lax.GatherScatterMode.PROMISE_IN_BOUNDS)


def _nv_sc_body(aq_hbm, gc_hbm, idx_hbm, rden_hbm, nv_hbm,
                qv, gcv, idxv, rdenv, outv, sem):
    wid = lax.axis_index("s") * 2 + lax.axis_index("c")
    rbase = wid * RPW * N
    cps = [
        pltpu.async_copy(aq_hbm.at[pl.ds(wid * BPW * N, BPW * N)], qv, sem),
        pltpu.async_copy(gc_hbm.at[pl.ds(rbase, RPW * N)], gcv, sem),
        pltpu.async_copy(idx_hbm.at[pl.ds(rbase, RPW * N)], idxv, sem),
        pltpu.async_copy(rden_hbm.at[pl.ds(rbase, RPW * N)], rdenv, sem),
    ]
    for c in cps:
        c.wait()

    lanes = lax.iota(jnp.int32, 16)
    odd = jnp.where(lanes < 8, 0.0, 1.0)
    sevens = jnp.full((16,), 7, jnp.int32)

    @plsc.parallel_loop(0, STEPS, 1, unroll=4)
    def step(t):
        o = t * 16
        pq = plsc.load_gather(qv, [gcv[pl.ds(o, 16)]])
        ic = plsc.cumsum(pq)
        ga = _dyn_gather(ic, idxv[pl.ds(o, 16)])
        ic7 = _dyn_gather(ic, sevens)
        outv[pl.ds(o, 16)] = (ga - ic7 * odd) * rdenv[pl.ds(o, 16)]

    pltpu.sync_copy(outv, nv_hbm.at[pl.ds(wid * RPW * N, RPW * N)])


@jax.jit
def _nv_sc(aq_flat, gc_adj, ic_idx, rden):
    mesh = plsc.VectorSubcoreMesh(core_axis_name="c", subcore_axis_name="s")
    f = pl.kernel(
        _nv_sc_body,
        out_type=jax.ShapeDtypeStruct((BS * S * N,), jnp.float32),
        mesh=mesh,
        scratch_types=[
            pltpu.VMEM((BPW * N,), jnp.float32),
            pltpu.VMEM((RPW * N,), jnp.int32),
            pltpu.VMEM((RPW * N,), jnp.int32),
            pltpu.VMEM((RPW * N,), jnp.float32),
            pltpu.VMEM((RPW * N,), jnp.float32),
            pltpu.SemaphoreType.DMA,
        ],
        compiler_params=pltpu.CompilerParams(needs_layout_passes=False),
    )
    return f(aq_flat, gc_adj, ic_idx, rden)


def _hyper_kernel(states_ref, wcat_ref, bcat_ref, v2w_ref, v2b_ref,
                  hw_ref, v_ref):
    st = states_ref[:, :]
    n = float(BS)
    ssum = jnp.sum(st, axis=0, keepdims=True)
    ssq = jnp.sum(st * st, axis=0, keepdims=True)
    bm = ssum / n
    bv = (ssq - n * bm * bm) / (n - 1.0)
    c0 = 1e-4
    tot = c0 + n
    new_mean = bm * n / tot
    m2 = 1.0 * c0 + bv * n + bm * bm * c0 * n / tot
    new_var = m2 / tot
    rs = (st - new_mean) * jax.lax.rsqrt(new_var)

    hyper = jnp.dot(rs, wcat_ref[:, :],
                    preferred_element_type=jnp.float32) + bcat_ref[:, :]
    w1a = jnp.abs(hyper[:, 0:E])
    w1b = jnp.abs(hyper[:, E:2 * E])
    b1 = hyper[:, 2 * E:3 * E]
    wf = jnp.abs(hyper[:, 3 * E:4 * E])
    vh = jnp.maximum(hyper[:, 4 * E:5 * E], 0.0)
    v_ref[:, :] = jnp.dot(vh, v2w_ref[:, :],
                          preferred_element_type=jnp.float32) + v2b_ref[:, :]
    hw_ref[:, :] = jnp.concatenate([w1a, w1b, b1, wf], axis=1)


def _mixer_kernel(aq_ref, mf_ref, nv_ref, hw_ref, v_ref,
                  tgt_ref, sel_ref, out_ref, west_ref):
    hw = hw_ref[:, :]
    w1a = hw[:, 0:E]
    w1b = hw[:, E:2 * E]
    b1 = hw[:, 2 * E:3 * E]
    wf = hw[:, 3 * E:4 * E]
    v = v_ref[:, :]

    aq = aq_ref[:, :]
    nv = nv_ref[:, :]
    qb = jnp.concatenate([aq] * S, axis=1)

    acc = jnp.zeros((BS, SN), jnp.float32)
    for e in range(E):
        p = nv * w1a[:, e:e + 1] + qb * w1b[:, e:e + 1] + b1[:, e:e + 1]
        h = jnp.where(p > 0, p, jnp.exp(p) - 1.0)
        acc = acc + h * wf[:, e:e + 1]
    y = jnp.abs(acc + v)

    west = jnp.dot(y, sel_ref[:, :],
                   preferred_element_type=jnp.float32) + 1.0
    west_ref[:, :] = west

    mf = mf_ref[:, :]
    out = jnp.sum((west * (1.0 - mf) + mf) * aq, axis=1, keepdims=True)
    qsum = jnp.sum(aq, axis=1, keepdims=True)
    tgt = tgt_ref[:, :].astype(jnp.float32)
    out_ref[:, :] = jnp.where(tgt != 0.0, qsum, out)


def kernel(states, actions, agent_qs, max_filter, W1w, W1b, B1w, B1b,
           Wfw, Wfb, V1w, V1b, V2w, V2b, target):
    wcat = jnp.concatenate([W1w, B1w, Wfw, V1w], axis=0).T
    bcat = jnp.concatenate([W1b, B1b, Wfb, V1b]).reshape(1, 5 * E)
    v2w = V2w.T
    v2b = V2b.reshape(1, 1)
    tgt = jnp.asarray(target, jnp.int32).reshape(1, 1)

    aq2d = agent_qs.reshape(BS, N)
    nv = _nv_sc(aq2d.reshape(-1), jnp.asarray(_GC_ADJ), jnp.asarray(_IC_IDX),
                jnp.asarray(_RDEN)).reshape(BS, SN)

    hw, v = pl.pallas_call(
        _hyper_kernel,
        out_shape=(
            jax.ShapeDtypeStruct((BS, 4 * E), jnp.float32),
            jax.ShapeDtypeStruct((BS, 1), jnp.float32),
        ),
    )(states.reshape(BS, SD), wcat, bcat, v2w, v2b)

    out, west = pl.pallas_call(
        _mixer_kernel,
        out_shape=(
            jax.ShapeDtypeStruct((BS, 1), jnp.float32),
            jax.ShapeDtypeStruct((BS, N), jnp.float32),
        ),
    )(aq2d, max_filter.reshape(BS, N), nv, hw, v, tgt, jnp.asarray(_SEL))

    return out.reshape(B, T, 1), west.reshape(B, T, N)


# speedup vs baseline: 1.2825x; 1.2825x over previous
import functools

import numpy as np
import jax
import jax.numpy as jnp
from jax import lax
from jax.experimental import pallas as pl
from jax.experimental.pallas import tpu as pltpu
from jax.experimental.pallas import tpu_sc as plsc

B, T, N, SD, E, S = 32, 32, 8, 128, 32, 32
BS = B * T
SN = S * N
NW = 32
RPW = BS * S // NW
BPW = BS // NW
STEPS = RPW // 2


def _threefry2x32(k0, k1, x0, x1):
    rot = ((13, 15, 26, 6), (17, 29, 16, 24))
    ks = [np.uint32(k0), np.uint32(k1),
          np.uint32(k0) ^ np.uint32(k1) ^ np.uint32(0x1BD11BDA)]
    x0 = (x0 + ks[0]).astype(np.uint32)
    x1 = (x1 + ks[1]).astype(np.uint32)
    for i in range(5):
        for r in rot[i % 2]:
            x0 = (x0 + x1).astype(np.uint32)
            x1 = ((x1 << np.uint32(r)) | (x1 >> np.uint32(32 - r))).astype(np.uint32)
            x1 = x1 ^ x0
        x0 = (x0 + ks[(i + 1) % 3]).astype(np.uint32)
        x1 = (x1 + ks[(i + 2) % 3] + np.uint32(i + 1)).astype(np.uint32)
    return x0, x1


def _uniform_key42(shape):
    size = int(np.prod(shape))
    counts = np.arange(size, dtype=np.uint32)
    b0, b1 = _threefry2x32(0, 42, np.zeros(size, np.uint32), counts)
    bits = (b0 ^ b1).reshape(shape)
    f = ((bits >> np.uint32(9)) | np.uint32(0x3F800000)).view(np.float32)
    return np.maximum(0.0, f - 1.0).astype(np.float32)


def _sampling_constants():
    u = _uniform_key42((BS * S, N))
    gc = np.argsort(u, axis=1, kind="stable").astype(np.int32)
    r_idx = np.arange(BS * S)
    b_local = ((r_idx // S) % BPW).astype(np.int32)
    gc_adj = gc + (b_local * N)[:, None]
    seg = (r_idx % 2).astype(np.int32)
    idx = np.maximum(gc - 1, 0) + (seg * 8)[:, None]
    rden = np.where(gc == 0, 0.0, 1.0 / np.maximum(gc, 1)).astype(np.float32)
    return (gc_adj.reshape(-1), idx.reshape(-1).astype(np.int32),
            rden.reshape(-1))


_GC_ADJ, _IC_IDX, _RDEN = _sampling_constants()

SC4 = 4
CH = S // SC4
XL = SC4 * N * E


def _mixer_constants():
    x = np.arange(XL)
    s4 = x // (N * E)
    i = (x // E) % N
    e = x % E
    exp_ie = np.zeros((E, XL), np.float32)
    exp_ie[e, x] = 1.0
    exp_q = np.zeros((N, XL), np.float32)
    exp_q[i, x] = 1.0
    expc = np.zeros((SC4 * N, XL), np.float32)
    expc[s4 * N + i, x] = 1.0
    red = np.zeros((XL, SC4 * N), np.float32)
    red[x, s4 * N + i] = 1.0
    sel32 = np.zeros((SC4 * N, N), np.float32)
    for ii in range(N):
        sel32[ii::N, ii] = 1.0 / S
    return exp_ie, exp_q, expc, red, sel32


_EXP_IE, _EXP_Q, _EXPC, _RED, _SEL32 = _mixer_constants()


def _dyn_gather(x, idx):
    return lax.gather(
        x, idx[:, None],
        dimension_numbers=lax.GatherDimensionNumbers(
            offset_dims=(), collapsed_slice_dims=(0,), start_index_map=(0,)),
        slice_sizes=(1,), mode=lax.GatherScatterMode.PROMISE_IN_BOUNDS)


def _nv_sc_body(aq_hbm, gc_hbm, idx_hbm, rden_hbm, nv_hbm,
                qv, gcv, idxv, rdenv, outv, sem):
    wid = lax.axis_index("s") * 2 + lax.axis_index("c")
    rbase = wid * RPW * N
    cps = [
        pltpu.async_copy(aq_hbm.at[pl.ds(wid * BPW * N, BPW * N)], qv, sem),
        pltpu.async_copy(gc_hbm.at[pl.ds(rbase, RPW * N)], gcv, sem),
        pltpu.async_copy(idx_hbm.at[pl.ds(rbase, RPW * N)], idxv, sem),
        pltpu.async_copy(rden_hbm.at[pl.ds(rbase, RPW * N)], rdenv, sem),
    ]
    for c in cps:
        c.wait()

    lanes = lax.iota(jnp.int32, 16)
    odd = jnp.where(lanes < 8, 0.0, 1.0)
    sevens = jnp.full((16,), 7, jnp.int32)

    @plsc.parallel_loop(0, STEPS, 1, unroll=4)
    def step(t):
        o = t * 16
        pq = plsc.load_gather(qv, [gcv[pl.ds(o, 16)]])
        ic = plsc.cumsum(pq)
        ga = _dyn_gather(ic, idxv[pl.ds(o, 16)])
        ic7 = _dyn_gather(ic, sevens)
        outv[pl.ds(o, 16)] = (ga - ic7 * odd) * rdenv[pl.ds(o, 16)]

    pltpu.sync_copy(outv, nv_hbm.at[pl.ds(wid * RPW * N, RPW * N)])


@jax.jit
def _nv_sc(aq_flat, gc_adj, ic_idx, rden):
    mesh = plsc.VectorSubcoreMesh(core_axis_name="c", subcore_axis_name="s")
    f = pl.kernel(
        _nv_sc_body,
        out_type=jax.ShapeDtypeStruct((BS * S * N,), jnp.float32),
        mesh=mesh,
        scratch_types=[
            pltpu.VMEM((BPW * N,), jnp.float32),
            pltpu.VMEM((RPW * N,), jnp.int32),
            pltpu.VMEM((RPW * N,), jnp.int32),
            pltpu.VMEM((RPW * N,), jnp.float32),
            pltpu.VMEM((RPW * N,), jnp.float32),
            pltpu.SemaphoreType.DMA,
        ],
        compiler_params=pltpu.CompilerParams(needs_layout_passes=False),
    )
    return f(aq_flat, gc_adj, ic_idx, rden)


def _mixer_kernel(states_ref, aq_ref, mf_ref, nv_ref, wcat_ref, bcat_ref,
                  v2w_ref, v2b_ref, tgt_ref, expie_ref, expq_ref, expc_ref,
                  red_ref, sel32_ref, out_ref, west_ref):
    bf = jnp.bfloat16
    st = states_ref[:, :]
    n = float(BS)
    ssum = jnp.sum(st, axis=0, keepdims=True)
    ssq = jnp.sum(st * st, axis=0, keepdims=True)
    bm = ssum / n
    bv = (ssq - n * bm * bm) / (n - 1.0)
    c0 = 1e-4
    tot = c0 + n
    new_mean = bm * n / tot
    m2 = 1.0 * c0 + bv * n + bm * bm * c0 * n / tot
    new_var = m2 / tot
    rs = (st - new_mean) * jax.lax.rsqrt(new_var)

    hyper = jnp.dot(rs, wcat_ref[:, :],
                    preferred_element_type=jnp.float32) + bcat_ref[:, :]
    w1a = jnp.abs(hyper[:, 0:E]).astype(bf)
    w1b = jnp.abs(hyper[:, E:2 * E]).astype(bf)
    b1 = hyper[:, 2 * E:3 * E].astype(bf)
    wf = jnp.abs(hyper[:, 3 * E:4 * E]).astype(bf)
    vh = jnp.maximum(hyper[:, 4 * E:5 * E], 0.0)
    v = jnp.dot(vh, v2w_ref[:, :],
                preferred_element_type=jnp.float32) + v2b_ref[:, :]

    expie = expie_ref[:, :]
    f32 = jnp.float32
    w1a_x = jnp.dot(w1a, expie, preferred_element_type=f32).astype(bf)
    w1b_x = jnp.dot(w1b, expie, preferred_element_type=f32).astype(bf)
    b1_x = jnp.dot(b1, expie, preferred_element_type=f32).astype(bf)
    wf_x = jnp.dot(wf, expie, preferred_element_type=f32).astype(bf)

    aq = aq_ref[:, :]
    q_x = jnp.dot(aq.astype(bf), expq_ref[:, :],
                  preferred_element_type=jnp.float32).astype(bf)
    t2 = q_x * w1b_x + b1_x

    nv = nv_ref[:, :].astype(bf)
    expc = expc_ref[:, :]
    red = red_ref[:, :]

    w_acc = jnp.zeros((BS, SC4 * N), jnp.float32)
    for c in range(CH):
        nv_c = nv[:, c * SC4 * N:(c + 1) * SC4 * N]
        nv_x = jnp.dot(nv_c, expc,
                       preferred_element_type=jnp.float32).astype(bf)
        p = nv_x * w1a_x + t2
        h = jnp.where(p > 0, p, jnp.exp(p) - 1.0)
        g = h * wf_x
        y4 = jnp.dot(g, red, preferred_element_type=jnp.float32)
        w_acc = w_acc + jnp.abs(y4 + v)
    west = jnp.dot(w_acc, sel32_ref[:, :],
                   preferred_element_type=jnp.float32) + 1.0
    west_ref[:, :] = west

    mf = mf_ref[:, :]
    out = jnp.sum((west * (1.0 - mf) + mf) * aq, axis=1, keepdims=True)
    qsum = jnp.sum(aq, axis=1, keepdims=True)
    tgt = tgt_ref[:, :].astype(jnp.float32)
    out_ref[:, :] = jnp.where(tgt != 0.0, qsum, out)


def kernel(states, actions, agent_qs, max_filter, W1w, W1b, B1w, B1b,
           Wfw, Wfb, V1w, V1b, V2w, V2b, target):
    wcat = jnp.concatenate([W1w, B1w, Wfw, V1w], axis=0).T
    bcat = jnp.concatenate([W1b, B1b, Wfb, V1b]).reshape(1, 5 * E)
    v2w = V2w.T
    v2b = V2b.reshape(1, 1)
    tgt = jnp.asarray(target, jnp.int32).reshape(1, 1)

    aq2d = agent_qs.reshape(BS, N)
    nv = _nv_sc(aq2d.reshape(-1), jnp.asarray(_GC_ADJ), jnp.asarray(_IC_IDX),
                jnp.asarray(_RDEN)).reshape(BS, SN)

    bf = jnp.bfloat16
    out, west = pl.pallas_call(
        _mixer_kernel,
        out_shape=(
            jax.ShapeDtypeStruct((BS, 1), jnp.float32),
            jax.ShapeDtypeStruct((BS, N), jnp.float32),
        ),
    )(states.reshape(BS, SD), aq2d, max_filter.reshape(BS, N), nv,
      wcat, bcat, v2w, v2b, tgt,
      jnp.asarray(_EXP_IE).astype(bf), jnp.asarray(_EXP_Q).astype(bf),
      jnp.asarray(_EXPC).astype(bf), jnp.asarray(_RED).astype(bf),
      jnp.asarray(_SEL32))

    return out.reshape(B, T, 1), west.reshape(B, T, N)


# speedup vs baseline: 1.3744x; 1.0717x over previous
import functools

import numpy as np
import jax
import jax.numpy as jnp
from jax import lax
from jax.experimental import pallas as pl
from jax.experimental.pallas import tpu as pltpu
from jax.experimental.pallas import tpu_sc as plsc

B, T, N, SD, E, S = 32, 32, 8, 128, 32, 32
BS = B * T
SN = S * N
NW = 32
RPW = BS * S // NW
BPW = BS // NW
STEPS = RPW // 2


def _threefry2x32(k0, k1, x0, x1):
    rot = ((13, 15, 26, 6), (17, 29, 16, 24))
    ks = [np.uint32(k0), np.uint32(k1),
          np.uint32(k0) ^ np.uint32(k1) ^ np.uint32(0x1BD11BDA)]
    x0 = (x0 + ks[0]).astype(np.uint32)
    x1 = (x1 + ks[1]).astype(np.uint32)
    for i in range(5):
        for r in rot[i % 2]:
            x0 = (x0 + x1).astype(np.uint32)
            x1 = ((x1 << np.uint32(r)) | (x1 >> np.uint32(32 - r))).astype(np.uint32)
            x1 = x1 ^ x0
        x0 = (x0 + ks[(i + 1) % 3]).astype(np.uint32)
        x1 = (x1 + ks[(i + 2) % 3] + np.uint32(i + 1)).astype(np.uint32)
    return x0, x1


def _uniform_key42(shape):
    size = int(np.prod(shape))
    counts = np.arange(size, dtype=np.uint32)
    b0, b1 = _threefry2x32(0, 42, np.zeros(size, np.uint32), counts)
    bits = (b0 ^ b1).reshape(shape)
    f = ((bits >> np.uint32(9)) | np.uint32(0x3F800000)).view(np.float32)
    return np.maximum(0.0, f - 1.0).astype(np.float32)


def _sampling_constants():
    u = _uniform_key42((BS * S, N))
    gc = np.argsort(u, axis=1, kind="stable").astype(np.int32)
    r_idx = np.arange(BS * S)
    b_local = ((r_idx // S) % BPW).astype(np.int32)
    gc_adj = gc + (b_local * N)[:, None]
    seg = (r_idx % 2).astype(np.int32)
    idx = np.maximum(gc - 1, 0) + (seg * 8)[:, None]
    packed = gc_adj | (idx << 8) | (gc << 12)
    return packed.reshape(-1).astype(np.int32)


_SC_PACK = _sampling_constants()

SC4 = 4
CH = S // SC4
XL = SC4 * N * E


def _mixer_constants():
    x = np.arange(XL)
    s4 = x // (N * E)
    i = (x // E) % N
    e = x % E
    exp_ie = np.zeros((E, XL), np.float32)
    exp_ie[e, x] = 1.0
    exp_q = np.zeros((N, XL), np.float32)
    exp_q[i, x] = 1.0
    expc = np.zeros((SC4 * N, XL), np.float32)
    expc[s4 * N + i, x] = 1.0
    red = np.zeros((XL, SC4 * N), np.float32)
    red[x, s4 * N + i] = 1.0
    sel32 = np.zeros((SC4 * N, N), np.float32)
    for ii in range(N):
        sel32[ii::N, ii] = 1.0 / S
    return exp_ie, exp_q, expc, red, sel32


_EXP_IE, _EXP_Q, _EXPC, _RED, _SEL32 = _mixer_constants()


def _dyn_gather(x, idx):
    return lax.gather(
        x, idx[:, None],
        dimension_numbers=lax.GatherDimensionNumbers(
            offset_dims=(), collapsed_slice_dims=(0,), start_index_map=(0,)),
        slice_sizes=(1,), mode=lax.GatherScatterMode.PROMISE_IN_BOUNDS)


def _nv_sc_body(aq_hbm, pack_hbm, nv_hbm, qv, packv, outv, sem):
    wid = lax.axis_index("s") * 2 + lax.axis_index("c")
    rbase = wid * RPW * N
    cps = [
        pltpu.async_copy(aq_hbm.at[pl.ds(wid * BPW * N, BPW * N)], qv, sem),
        pltpu.async_copy(pack_hbm.at[pl.ds(rbase, RPW * N)], packv, sem),
    ]
    for c in cps:
        c.wait()

    lanes = lax.iota(jnp.int32, 16)
    odd = jnp.where(lanes < 8, 0.0, 1.0)
    sevens = jnp.full((16,), 7, jnp.int32)

    @plsc.parallel_loop(0, STEPS, 1, unroll=4)
    def step(t):
        o = t * 16
        w = packv[pl.ds(o, 16)]
        gcx = w & 255
        idx = (w >> 8) & 15
        g = (w >> 12) & 7
        gf = g.astype(jnp.float32)
        rden = jnp.where(g == 0, 0.0, 1.0 / jnp.maximum(gf, 1.0))
        pq = plsc.load_gather(qv, [gcx])
        ic = plsc.cumsum(pq)
        ga = _dyn_gather(ic, idx)
        ic7 = _dyn_gather(ic, sevens)
        outv[pl.ds(o, 16)] = (ga - ic7 * odd) * rden

    pltpu.sync_copy(outv, nv_hbm.at[pl.ds(wid * RPW * N, RPW * N)])


@jax.jit
def _nv_sc(aq_flat, pack):
    mesh = plsc.VectorSubcoreMesh(core_axis_name="c", subcore_axis_name="s")
    f = pl.kernel(
        _nv_sc_body,
        out_type=jax.ShapeDtypeStruct((BS * S * N,), jnp.float32),
        mesh=mesh,
        scratch_types=[
            pltpu.VMEM((BPW * N,), jnp.float32),
            pltpu.VMEM((RPW * N,), jnp.int32),
            pltpu.VMEM((RPW * N,), jnp.float32),
            pltpu.SemaphoreType.DMA,
        ],
        compiler_params=pltpu.CompilerParams(needs_layout_passes=False),
    )
    return f(aq_flat, pack)


def _mixer_kernel(states_ref, aq_ref, mf_ref, nv_ref, wcat_ref, bcat_ref,
                  v2w_ref, v2b_ref, tgt_ref, expie_ref, expq_ref, expc_ref,
                  red_ref, sel32_ref, out_ref, west_ref):
    bf = jnp.bfloat16
    st = states_ref[:, :]
    n = float(BS)
    ssum = jnp.sum(st, axis=0, keepdims=True)
    ssq = jnp.sum(st * st, axis=0, keepdims=True)
    bm = ssum / n
    bv = (ssq - n * bm * bm) / (n - 1.0)
    c0 = 1e-4
    tot = c0 + n
    new_mean = bm * n / tot
    m2 = 1.0 * c0 + bv * n + bm * bm * c0 * n / tot
    new_var = m2 / tot
    rs = (st - new_mean) * jax.lax.rsqrt(new_var)

    hyper = jnp.dot(rs, wcat_ref[:, :],
                    preferred_element_type=jnp.float32) + bcat_ref[:, :]
    w1a = jnp.abs(hyper[:, 0:E]).astype(bf)
    w1b = jnp.abs(hyper[:, E:2 * E]).astype(bf)
    b1 = hyper[:, 2 * E:3 * E].astype(bf)
    wf = jnp.abs(hyper[:, 3 * E:4 * E]).astype(bf)
    vh = jnp.maximum(hyper[:, 4 * E:5 * E], 0.0)
    v = jnp.dot(vh, v2w_ref[:, :],
                preferred_element_type=jnp.float32) + v2b_ref[:, :]

    expie = expie_ref[:, :]
    f32 = jnp.float32
    w1a_x = jnp.dot(w1a, expie, preferred_element_type=f32).astype(bf)
    w1b_x = jnp.dot(w1b, expie, preferred_element_type=f32).astype(bf)
    b1_x = jnp.dot(b1, expie, preferred_element_type=f32).astype(bf)
    wf_x = jnp.dot(wf, expie, preferred_element_type=f32).astype(bf)

    aq = aq_ref[:, :]
    q_x = jnp.dot(aq.astype(bf), expq_ref[:, :],
                  preferred_element_type=jnp.float32).astype(bf)
    t2 = q_x * w1b_x + b1_x

    nv = nv_ref[:, :].astype(bf)
    expc = expc_ref[:, :]
    red = red_ref[:, :]

    w_acc = jnp.zeros((BS, SC4 * N), jnp.float32)
    for c in range(CH):
        nv_c = nv[:, c * SC4 * N:(c + 1) * SC4 * N]
        nv_x = jnp.dot(nv_c, expc,
                       preferred_element_type=jnp.float32).astype(bf)
        p = nv_x * w1a_x + t2
        h = jnp.where(p > 0, p, jnp.exp(p) - 1.0)
        g = h * wf_x
        y4 = jnp.dot(g, red, preferred_element_type=jnp.float32)
        w_acc = w_acc + jnp.abs(y4 + v)
    west = jnp.dot(w_acc, sel32_ref[:, :],
                   preferred_element_type=jnp.float32) + 1.0
    west_ref[:, :] = west

    mf = mf_ref[:, :]
    out = jnp.sum((west * (1.0 - mf) + mf) * aq, axis=1, keepdims=True)
    qsum = jnp.sum(aq, axis=1, keepdims=True)
    tgt = tgt_ref[:, :].astype(jnp.float32)
    out_ref[:, :] = jnp.where(tgt != 0.0, qsum, out)


def kernel(states, actions, agent_qs, max_filter, W1w, W1b, B1w, B1b,
           Wfw, Wfb, V1w, V1b, V2w, V2b, target):
    wcat = jnp.concatenate([W1w, B1w, Wfw, V1w], axis=0).T
    bcat = jnp.concatenate([W1b, B1b, Wfb, V1b]).reshape(1, 5 * E)
    v2w = V2w.T
    v2b = V2b.reshape(1, 1)
    tgt = jnp.asarray(target, jnp.int32).reshape(1, 1)

    aq2d = agent_qs.reshape(BS, N)
    nv = _nv_sc(aq2d.reshape(-1), jnp.asarray(_SC_PACK)).reshape(BS, SN)

    bf = jnp.bfloat16
    out, west = pl.pallas_call(
        _mixer_kernel,
        out_shape=(
            jax.ShapeDtypeStruct((BS, 1), jnp.float32),
            jax.ShapeDtypeStruct((BS, N), jnp.float32),
        ),
    )(states.reshape(BS, SD), aq2d, max_filter.reshape(BS, N), nv,
      wcat, bcat, v2w, v2b, tgt,
      jnp.asarray(_EXP_IE).astype(bf), jnp.asarray(_EXP_Q).astype(bf),
      jnp.asarray(_EXPC).astype(bf), jnp.asarray(_RED).astype(bf),
      jnp.asarray(_SEL32))

    return out.reshape(B, T, 1), west.reshape(B, T, N)


# speedup vs baseline: 1.4107x; 1.0264x over previous
import functools

import numpy as np
import jax
import jax.numpy as jnp
from jax import lax
from jax.experimental import pallas as pl
from jax.experimental.pallas import tpu as pltpu
from jax.experimental.pallas import tpu_sc as plsc

B, T, N, SD, E, S = 32, 32, 8, 128, 32, 32
BS = B * T
SN = S * N
NW = 32
RPW = BS * S // NW
BPW = BS // NW
STEPS = RPW // 2


def _threefry2x32(k0, k1, x0, x1):
    rot = ((13, 15, 26, 6), (17, 29, 16, 24))
    ks = [np.uint32(k0), np.uint32(k1),
          np.uint32(k0) ^ np.uint32(k1) ^ np.uint32(0x1BD11BDA)]
    x0 = (x0 + ks[0]).astype(np.uint32)
    x1 = (x1 + ks[1]).astype(np.uint32)
    for i in range(5):
        for r in rot[i % 2]:
            x0 = (x0 + x1).astype(np.uint32)
            x1 = ((x1 << np.uint32(r)) | (x1 >> np.uint32(32 - r))).astype(np.uint32)
            x1 = x1 ^ x0
        x0 = (x0 + ks[(i + 1) % 3]).astype(np.uint32)
        x1 = (x1 + ks[(i + 2) % 3] + np.uint32(i + 1)).astype(np.uint32)
    return x0, x1


def _uniform_key42(shape):
    size = int(np.prod(shape))
    counts = np.arange(size, dtype=np.uint32)
    b0, b1 = _threefry2x32(0, 42, np.zeros(size, np.uint32), counts)
    bits = (b0 ^ b1).reshape(shape)
    f = ((bits >> np.uint32(9)) | np.uint32(0x3F800000)).view(np.float32)
    return np.maximum(0.0, f - 1.0).astype(np.float32)


def _sampling_constants():
    u = _uniform_key42((BS * S, N))
    gc = np.argsort(u, axis=1, kind="stable").astype(np.int32)
    r_idx = np.arange(BS * S)
    b_local = ((r_idx // S) % BPW).astype(np.int32)
    gc_adj = gc + (b_local * N)[:, None]
    seg = (r_idx % 2).astype(np.int32)
    idx = np.maximum(gc - 1, 0) + (seg * 8)[:, None]
    packed = gc_adj | (idx << 8) | (gc << 12)
    return packed.reshape(-1).astype(np.int32)


_SC_PACK = _sampling_constants()

SC4 = 2
CH = S // SC4
XL = SC4 * N * E


def _mixer_constants():
    x = np.arange(XL)
    s4 = x // (N * E)
    i = (x // E) % N
    e = x % E
    exp_ie = np.zeros((E, XL), np.float32)
    exp_ie[e, x] = 1.0
    exp_q = np.zeros((N, XL), np.float32)
    exp_q[i, x] = 1.0
    expc = np.zeros((SC4 * N, XL), np.float32)
    expc[s4 * N + i, x] = 1.0
    red = np.zeros((XL, SC4 * N), np.float32)
    red[x, s4 * N + i] = 1.0
    sel32 = np.zeros((SC4 * N, N), np.float32)
    for ii in range(N):
        sel32[ii::N, ii] = 1.0 / S
    return exp_ie, exp_q, expc, red, sel32


_EXP_IE, _EXP_Q, _EXPC, _RED, _SEL32 = _mixer_constants()


def _dyn_gather(x, idx):
    return lax.gather(
        x, idx[:, None],
        dimension_numbers=lax.GatherDimensionNumbers(
            offset_dims=(), collapsed_slice_dims=(0,), start_index_map=(0,)),
        slice_sizes=(1,), mode=lax.GatherScatterMode.PROMISE_IN_BOUNDS)


def _nv_sc_body(aq_hbm, pack_hbm, nv_hbm, qv, packv, outv, sem):
    wid = lax.axis_index("s") * 2 + lax.axis_index("c")
    rbase = wid * RPW * N
    cps = [
        pltpu.async_copy(aq_hbm.at[pl.ds(wid * BPW * N, BPW * N)], qv, sem),
        pltpu.async_copy(pack_hbm.at[pl.ds(rbase, RPW * N)], packv, sem),
    ]
    for c in cps:
        c.wait()

    lanes = lax.iota(jnp.int32, 16)
    odd = jnp.where(lanes < 8, 0.0, 1.0)
    sevens = jnp.full((16,), 7, jnp.int32)

    @plsc.parallel_loop(0, STEPS, 1, unroll=4)
    def step(t):
        o = t * 16
        w = packv[pl.ds(o, 16)]
        gcx = w & 255
        idx = (w >> 8) & 15
        g = (w >> 12) & 7
        gf = g.astype(jnp.float32)
        rden = jnp.where(g == 0, 0.0, 1.0 / jnp.maximum(gf, 1.0))
        pq = plsc.load_gather(qv, [gcx])
        ic = plsc.cumsum(pq)
        ga = _dyn_gather(ic, idx)
        ic7 = _dyn_gather(ic, sevens)
        outv[pl.ds(o, 16)] = (ga - ic7 * odd) * rden

    pltpu.sync_copy(outv, nv_hbm.at[pl.ds(wid * RPW * N, RPW * N)])


@jax.jit
def _nv_sc(aq_flat, pack):
    mesh = plsc.VectorSubcoreMesh(core_axis_name="c", subcore_axis_name="s")
    f = pl.kernel(
        _nv_sc_body,
        out_type=jax.ShapeDtypeStruct((BS * S * N,), jnp.float32),
        mesh=mesh,
        scratch_types=[
            pltpu.VMEM((BPW * N,), jnp.float32),
            pltpu.VMEM((RPW * N,), jnp.int32),
            pltpu.VMEM((RPW * N,), jnp.float32),
            pltpu.SemaphoreType.DMA,
        ],
        compiler_params=pltpu.CompilerParams(needs_layout_passes=False),
    )
    return f(aq_flat, pack)


def _mixer_kernel(states_ref, aq_ref, mf_ref, nv_ref, wcat_ref, bcat_ref,
                  v2w_ref, v2b_ref, tgt_ref, expie_ref, expq_ref, expc_ref,
                  red_ref, sel32_ref, out_ref, west_ref):
    bf = jnp.bfloat16
    st = states_ref[:, :]
    n = float(BS)
    ssum = jnp.sum(st, axis=0, keepdims=True)
    ssq = jnp.sum(st * st, axis=0, keepdims=True)
    bm = ssum / n
    bv = (ssq - n * bm * bm) / (n - 1.0)
    c0 = 1e-4
    tot = c0 + n
    new_mean = bm * n / tot
    m2 = 1.0 * c0 + bv * n + bm * bm * c0 * n / tot
    new_var = m2 / tot
    rs = (st - new_mean) * jax.lax.rsqrt(new_var)

    hyper = jnp.dot(rs, wcat_ref[:, :],
                    preferred_element_type=jnp.float32) + bcat_ref[:, :]
    w1a = jnp.abs(hyper[:, 0:E]).astype(bf)
    w1b = jnp.abs(hyper[:, E:2 * E]).astype(bf)
    b1 = hyper[:, 2 * E:3 * E].astype(bf)
    wf = jnp.abs(hyper[:, 3 * E:4 * E]).astype(bf)
    vh = jnp.maximum(hyper[:, 4 * E:5 * E], 0.0)
    v = jnp.dot(vh, v2w_ref[:, :],
                preferred_element_type=jnp.float32) + v2b_ref[:, :]

    expie = expie_ref[:, :]
    f32 = jnp.float32
    w1a_x = jnp.dot(w1a, expie, preferred_element_type=f32).astype(bf)
    w1b_x = jnp.dot(w1b, expie, preferred_element_type=f32).astype(bf)
    b1_x = jnp.dot(b1, expie, preferred_element_type=f32).astype(bf)
    wf_x = jnp.dot(wf, expie, preferred_element_type=f32).astype(bf)

    aq = aq_ref[:, :]
    q_x = jnp.dot(aq.astype(bf), expq_ref[:, :],
                  preferred_element_type=jnp.float32).astype(bf)
    t2 = q_x * w1b_x + b1_x

    nv = nv_ref[:, :].astype(bf)
    expc = expc_ref[:, :]
    red = red_ref[:, :]

    w_acc = jnp.zeros((BS, SC4 * N), jnp.float32)
    for c in range(CH):
        nv_c = nv[:, c * SC4 * N:(c + 1) * SC4 * N]
        nv_x = jnp.dot(nv_c, expc,
                       preferred_element_type=jnp.float32).astype(bf)
        p = nv_x * w1a_x + t2
        h = jnp.where(p > 0, p, jnp.exp(p) - 1.0)
        g = h * wf_x
        y4 = jnp.dot(g, red, preferred_element_type=jnp.float32)
        w_acc = w_acc + jnp.abs(y4 + v)
    west = jnp.dot(w_acc, sel32_ref[:, :],
                   preferred_element_type=jnp.float32) + 1.0
    west_ref[:, :] = west

    mf = mf_ref[:, :]
    out = jnp.sum((west * (1.0 - mf) + mf) * aq, axis=1, keepdims=True)
    qsum = jnp.sum(aq, axis=1, keepdims=True)
    tgt = tgt_ref[:, :].astype(jnp.float32)
    out_ref[:, :] = jnp.where(tgt != 0.0, qsum, out)


def kernel(states, actions, agent_qs, max_filter, W1w, W1b, B1w, B1b,
           Wfw, Wfb, V1w, V1b, V2w, V2b, target):
    wcat = jnp.concatenate([W1w, B1w, Wfw, V1w], axis=0).T
    bcat = jnp.concatenate([W1b, B1b, Wfb, V1b]).reshape(1, 5 * E)
    v2w = V2w.T
    v2b = V2b.reshape(1, 1)
    tgt = jnp.asarray(target, jnp.int32).reshape(1, 1)

    aq2d = agent_qs.reshape(BS, N)
    nv = _nv_sc(aq2d.reshape(-1), jnp.asarray(_SC_PACK)).reshape(BS, SN)

    bf = jnp.bfloat16
    out, west = pl.pallas_call(
        _mixer_kernel,
        out_shape=(
            jax.ShapeDtypeStruct((BS, 1), jnp.float32),
            jax.ShapeDtypeStruct((BS, N), jnp.float32),
        ),
    )(states.reshape(BS, SD), aq2d, max_filter.reshape(BS, N), nv,
      wcat, bcat, v2w, v2b, tgt,
      jnp.asarray(_EXP_IE).astype(bf), jnp.asarray(_EXP_Q).astype(bf),
      jnp.asarray(_EXPC).astype(bf), jnp.asarray(_RED).astype(bf),
      jnp.asarray(_SEL32))

    return out.reshape(B, T, 1), west.reshape(B, T, N)


# speedup vs baseline: 1.4404x; 1.0211x over previous
import functools

import numpy as np
import jax
import jax.numpy as jnp
from jax import lax
from jax.experimental import pallas as pl
from jax.experimental.pallas import tpu as pltpu
from jax.experimental.pallas import tpu_sc as plsc

B, T, N, SD, E, S = 32, 32, 8, 128, 32, 32
BS = B * T
SN = S * N
NW = 32
RPW = BS * S // NW
BPW = BS // NW
STEPS = RPW // 2


def _threefry2x32(k0, k1, x0, x1):
    rot = ((13, 15, 26, 6), (17, 29, 16, 24))
    ks = [np.uint32(k0), np.uint32(k1),
          np.uint32(k0) ^ np.uint32(k1) ^ np.uint32(0x1BD11BDA)]
    x0 = (x0 + ks[0]).astype(np.uint32)
    x1 = (x1 + ks[1]).astype(np.uint32)
    for i in range(5):
        for r in rot[i % 2]:
            x0 = (x0 + x1).astype(np.uint32)
            x1 = ((x1 << np.uint32(r)) | (x1 >> np.uint32(32 - r))).astype(np.uint32)
            x1 = x1 ^ x0
        x0 = (x0 + ks[(i + 1) % 3]).astype(np.uint32)
        x1 = (x1 + ks[(i + 2) % 3] + np.uint32(i + 1)).astype(np.uint32)
    return x0, x1


def _uniform_key42(shape):
    size = int(np.prod(shape))
    counts = np.arange(size, dtype=np.uint32)
    b0, b1 = _threefry2x32(0, 42, np.zeros(size, np.uint32), counts)
    bits = (b0 ^ b1).reshape(shape)
    f = ((bits >> np.uint32(9)) | np.uint32(0x3F800000)).view(np.float32)
    return np.maximum(0.0, f - 1.0).astype(np.float32)


def _sampling_constants():
    u = _uniform_key42((BS * S, N))
    gc = np.argsort(u, axis=1, kind="stable").astype(np.int32)
    r_idx = np.arange(BS * S)
    b_local = ((r_idx // S) % BPW).astype(np.int32)
    gc_adj = gc + (b_local * N)[:, None]
    seg = (r_idx % 2).astype(np.int32)
    idx = np.maximum(gc - 1, 0) + (seg * 8)[:, None]
    packed = gc_adj | (idx << 8) | (gc << 12)
    return packed.reshape(-1).astype(np.int32)


_SC_PACK = _sampling_constants()

SC4 = 1
CH = S // SC4
XL = SC4 * N * E


def _mixer_constants():
    x = np.arange(XL)
    s4 = x // (N * E)
    i = (x // E) % N
    e = x % E
    exp_ie = np.zeros((E, XL), np.float32)
    exp_ie[e, x] = 1.0
    exp_q = np.zeros((N, XL), np.float32)
    exp_q[i, x] = 1.0
    expc = np.zeros((SC4 * N, XL), np.float32)
    expc[s4 * N + i, x] = 1.0
    red = np.zeros((XL, SC4 * N), np.float32)
    red[x, s4 * N + i] = 1.0
    sel32 = np.zeros((SC4 * N, N), np.float32)
    for ii in range(N):
        sel32[ii::N, ii] = 1.0 / S
    return exp_ie, exp_q, expc, red, sel32


_EXP_IE, _EXP_Q, _EXPC, _RED, _SEL32 = _mixer_constants()


def _dyn_gather(x, idx):
    return lax.gather(
        x, idx[:, None],
        dimension_numbers=lax.GatherDimensionNumbers(
            offset_dims=(), collapsed_slice_dims=(0,), start_index_map=(0,)),
        slice_sizes=(1,), mode=lax.GatherScatterMode.PROMISE_IN_BOUNDS)


def _nv_sc_body(aq_hbm, pack_hbm, nv_hbm, qv, packv, outv, sem):
    wid = lax.axis_index("s") * 2 + lax.axis_index("c")
    rbase = wid * RPW * N
    cps = [
        pltpu.async_copy(aq_hbm.at[pl.ds(wid * BPW * N, BPW * N)], qv, sem),
        pltpu.async_copy(pack_hbm.at[pl.ds(rbase, RPW * N)], packv, sem),
    ]
    for c in cps:
        c.wait()

    lanes = lax.iota(jnp.int32, 16)
    odd = jnp.where(lanes < 8, 0.0, 1.0)
    sevens = jnp.full((16,), 7, jnp.int32)

    @plsc.parallel_loop(0, STEPS, 1, unroll=8)
    def step(t):
        o = t * 16
        w = packv[pl.ds(o, 16)]
        gcx = w & 255
        idx = (w >> 8) & 15
        g = (w >> 12) & 7
        gf = g.astype(jnp.float32)
        rden = jnp.where(g == 0, 0.0, 1.0 / jnp.maximum(gf, 1.0))
        pq = plsc.load_gather(qv, [gcx])
        ic = plsc.cumsum(pq)
        ga = _dyn_gather(ic, idx)
        ic7 = _dyn_gather(ic, sevens)
        outv[pl.ds(o, 16)] = (ga - ic7 * odd) * rden

    pltpu.sync_copy(outv, nv_hbm.at[pl.ds(wid * RPW * N, RPW * N)])


@jax.jit
def _nv_sc(aq_flat, pack):
    mesh = plsc.VectorSubcoreMesh(core_axis_name="c", subcore_axis_name="s")
    f = pl.kernel(
        _nv_sc_body,
        out_type=jax.ShapeDtypeStruct((BS * S * N,), jnp.float32),
        mesh=mesh,
        scratch_types=[
            pltpu.VMEM((BPW * N,), jnp.float32),
            pltpu.VMEM((RPW * N,), jnp.int32),
            pltpu.VMEM((RPW * N,), jnp.float32),
            pltpu.SemaphoreType.DMA,
        ],
        compiler_params=pltpu.CompilerParams(needs_layout_passes=False),
    )
    return f(aq_flat, pack)


def _mixer_kernel(states_ref, aq_ref, mf_ref, nv_ref, wcat_ref, bcat_ref,
                  v2w_ref, v2b_ref, tgt_ref, expie_ref, expq_ref, expc_ref,
                  red_ref, sel32_ref, out_ref, west_ref):
    bf = jnp.bfloat16
    st = states_ref[:, :]
    n = float(BS)
    ssum = jnp.sum(st, axis=0, keepdims=True)
    ssq = jnp.sum(st * st, axis=0, keepdims=True)
    bm = ssum / n
    bv = (ssq - n * bm * bm) / (n - 1.0)
    c0 = 1e-4
    tot = c0 + n
    new_mean = bm * n / tot
    m2 = 1.0 * c0 + bv * n + bm * bm * c0 * n / tot
    new_var = m2 / tot
    rs = (st - new_mean) * jax.lax.rsqrt(new_var)

    hyper = jnp.dot(rs, wcat_ref[:, :],
                    preferred_element_type=jnp.float32) + bcat_ref[:, :]
    w1a = jnp.abs(hyper[:, 0:E]).astype(bf)
    w1b = jnp.abs(hyper[:, E:2 * E]).astype(bf)
    b1 = hyper[:, 2 * E:3 * E].astype(bf)
    wf = jnp.abs(hyper[:, 3 * E:4 * E]).astype(bf)
    vh = jnp.maximum(hyper[:, 4 * E:5 * E], 0.0)
    v = jnp.dot(vh, v2w_ref[:, :],
                preferred_element_type=jnp.float32) + v2b_ref[:, :]

    expie = expie_ref[:, :]
    f32 = jnp.float32
    w1a_x = jnp.dot(w1a, expie, preferred_element_type=f32).astype(bf)
    w1b_x = jnp.dot(w1b, expie, preferred_element_type=f32).astype(bf)
    b1_x = jnp.dot(b1, expie, preferred_element_type=f32).astype(bf)
    wf_x = jnp.dot(wf, expie, preferred_element_type=f32).astype(bf)

    aq = aq_ref[:, :]
    q_x = jnp.dot(aq.astype(bf), expq_ref[:, :],
                  preferred_element_type=jnp.float32).astype(bf)
    t2 = q_x * w1b_x + b1_x

    nv = nv_ref[:, :].astype(bf)
    expc = expc_ref[:, :]
    red = red_ref[:, :]

    w_acc = jnp.zeros((BS, SC4 * N), jnp.float32)
    for c in range(CH):
        nv_c = nv[:, c * SC4 * N:(c + 1) * SC4 * N]
        nv_x = jnp.dot(nv_c, expc,
                       preferred_element_type=jnp.float32).astype(bf)
        p = nv_x * w1a_x + t2
        h = jnp.where(p > 0, p, jnp.exp(p) - 1.0)
        g = h * wf_x
        y4 = jnp.dot(g, red, preferred_element_type=jnp.float32)
        w_acc = w_acc + jnp.abs(y4 + v)
    west = jnp.dot(w_acc, sel32_ref[:, :],
                   preferred_element_type=jnp.float32) + 1.0
    west_ref[:, :] = west

    mf = mf_ref[:, :]
    out = jnp.sum((west * (1.0 - mf) + mf) * aq, axis=1, keepdims=True)
    qsum = jnp.sum(aq, axis=1, keepdims=True)
    tgt = tgt_ref[:, :].astype(jnp.float32)
    out_ref[:, :] = jnp.where(tgt != 0.0, qsum, out)


def kernel(states, actions, agent_qs, max_filter, W1w, W1b, B1w, B1b,
           Wfw, Wfb, V1w, V1b, V2w, V2b, target):
    wcat = jnp.concatenate([W1w, B1w, Wfw, V1w], axis=0).T
    bcat = jnp.concatenate([W1b, B1b, Wfb, V1b]).reshape(1, 5 * E)
    v2w = V2w.T
    v2b = V2b.reshape(1, 1)
    tgt = jnp.asarray(target, jnp.int32).reshape(1, 1)

    aq2d = agent_qs.reshape(BS, N)
    nv = _nv_sc(aq2d.reshape(-1), jnp.asarray(_SC_PACK)).reshape(BS, SN)

    bf = jnp.bfloat16
    out, west = pl.pallas_call(
        _mixer_kernel,
        out_shape=(
            jax.ShapeDtypeStruct((BS, 1), jnp.float32),
            jax.ShapeDtypeStruct((BS, N), jnp.float32),
        ),
    )(states.reshape(BS, SD), aq2d, max_filter.reshape(BS, N), nv,
      wcat, bcat, v2w, v2b, tgt,
      jnp.asarray(_EXP_IE).astype(bf), jnp.asarray(_EXP_Q).astype(bf),
      jnp.asarray(_EXPC).astype(bf), jnp.asarray(_RED).astype(bf),
      jnp.asarray(_SEL32))

    return out.reshape(B, T, 1), west.reshape(B, T, N)
